# baseline (speedup 1.0000x reference)
import jax
import jax.numpy as jnp
from jax import lax
from jax.experimental import pallas as pl
from jax.experimental.pallas import tpu as pltpu
from jax.experimental.pallas import tpu_sc as plsc

B = 1024
S = 200
M = 64
D = 64
NC = 2
NS = 16
NW = NC * NS
R = B // NW

S0, S1 = 104, 96


def _sort_body(lr_ref, lc_ref, ctf_ref, rank_ref, lsort_ref, isort_ref,
               cts_ref):
    lr = lr_ref[...]
    lc = lc_ref[...]
    ii = lax.broadcasted_iota(jnp.int32, (B, B), 0)
    jj = lax.broadcasted_iota(jnp.int32, (B, B), 1)
    before = (lr > lc) | ((lr == lc) & (jj < ii))
    rank_ref[...] = jnp.sum(before.astype(jnp.int32), axis=1, keepdims=True)
    before2 = (lc > lr) | ((lc == lr) & (ii < jj))
    rank_row = jnp.sum(before2.astype(jnp.int32), axis=0, keepdims=True)
    sel = ii == rank_row
    isort_ref[...] = jnp.sum(jnp.where(sel, jj, 0), axis=1, keepdims=True)
    lsort_ref[...] = jnp.sum(jnp.where(sel, lr, 0), axis=1, keepdims=True)
    p = sel.astype(jnp.float32)
    cts = jax.lax.dot(p, ctf_ref[...], precision=jax.lax.Precision.HIGHEST)
    cts_ref[...] = cts.astype(jnp.int32)


def _sort_tc(length_flat, ct):
    lr = length_flat.reshape(1, B)
    lc = length_flat.reshape(B, 1)
    ctf = ct.astype(jnp.float32)
    rank, lsorted, isort, cts = pl.pallas_call(
        _sort_body,
        out_shape=[
            jax.ShapeDtypeStruct((B, 1), jnp.int32),
            jax.ShapeDtypeStruct((B, 1), jnp.int32),
            jax.ShapeDtypeStruct((B, 1), jnp.int32),
            jax.ShapeDtypeStruct((B, S), jnp.int32),
        ],
    )(lr, lc, ctf)
    return rank.reshape(B), lsorted.reshape(B), isort, cts


def _sc_body(mi_hbm, ci_hbm, mat_hbm, tab_hbm, out_hbm,
             mi_v, ci_v, m0_v, m1_v, t0_v, t1_v, semm0, semm1, semt0, semt1):
    c = lax.axis_index("c")
    s = lax.axis_index("s")
    wid = s * NC + c
    base = wid * R
    pltpu.sync_copy(mi_hbm.at[pl.ds(base * 8, R * 8)], mi_v)
    pltpu.sync_copy(ci_hbm.at[pl.ds(base * S, R * S)], ci_v)

    mbufs = (m0_v, m1_v)
    tbufs = (t0_v, t1_v)
    msems = (semm0, semm1)
    tsems = (semt0, semt1)

    def issue(r, b):
        o = r * S
        pltpu.async_copy(mat_hbm.at[mi_v.at[pl.ds(r * 8, 1)]], mbufs[b],
                         msems[b])
        pltpu.async_copy(tab_hbm.at[ci_v.at[pl.ds(o, S0)]],
                         tbufs[b].at[pl.ds(0, S0)], tsems[b])
        pltpu.async_copy(tab_hbm.at[ci_v.at[pl.ds(o + S0, S1)]],
                         tbufs[b].at[pl.ds(S0, S1)], tsems[b])

    def drain(r, b):
        o = r * S
        pltpu.make_async_copy(mat_hbm.at[mi_v.at[pl.ds(r * 8, 1)]], mbufs[b],
                              msems[b]).wait()
        pltpu.make_async_copy(tab_hbm.at[ci_v.at[pl.ds(o, S0)]],
                              tbufs[b].at[pl.ds(0, S0)], tsems[b]).wait()
        pltpu.make_async_copy(tab_hbm.at[ci_v.at[pl.ds(o + S0, S1)]],
                              tbufs[b].at[pl.ds(S0, S1)], tsems[b]).wait()

    def write(r, b):
        k = base + r
        pltpu.sync_copy(mbufs[b].at[0, :, pl.ds(0, M)],
                        out_hbm.at[k, :, pl.ds(0, M)])
        pltpu.sync_copy(tbufs[b].at[:, pl.ds(0, D)],
                        out_hbm.at[k, :, pl.ds(M, D)])

    issue(0, 0)

    def pair(p, carry):
        r0 = 2 * p
        r1 = r0 + 1
        drain(r0, 0)
        issue(r1, 1)
        write(r0, 0)
        drain(r1, 1)

        @pl.when(p + 1 < R // 2)
        def _():
            issue(r0 + 2, 0)

        write(r1, 1)
        return carry

    lax.fori_loop(0, R // 2, pair, 0)


@jax.jit
def _run(matrix, length, core_terms, table):
    length_flat = length.reshape(B)
    ct = core_terms.reshape(B, S)
    idx_unsort, length_sorted, isort, cts = _sort_tc(length_flat, ct)

    mi_1d = jnp.pad(isort, ((0, 0), (0, 7))).reshape(B * 8)
    ci_1d = cts.reshape(B * S)
    mat_pad = jnp.pad(matrix.reshape(B, S, M), ((0, 0), (0, 0), (0, 128 - M)))
    tab_pad = jnp.pad(table, ((0, 0), (0, 128 - D)))

    mesh = plsc.VectorSubcoreMesh(core_axis_name="c", subcore_axis_name="s")
    x = pl.kernel(
        _sc_body,
        mesh=mesh,
        out_type=jax.ShapeDtypeStruct((B, S, M + D), jnp.float32),
        compiler_params=pltpu.CompilerParams(use_tc_tiling_on_sc=False),
        scratch_types=[
            pltpu.VMEM((R * 8,), jnp.int32),
            pltpu.VMEM((R * S,), jnp.int32),
            pltpu.VMEM((1, S, 128), jnp.float32),
            pltpu.VMEM((1, S, 128), jnp.float32),
            pltpu.VMEM((S, 128), jnp.float32),
            pltpu.VMEM((S, 128), jnp.float32),
            pltpu.SemaphoreType.DMA,
            pltpu.SemaphoreType.DMA,
            pltpu.SemaphoreType.DMA,
            pltpu.SemaphoreType.DMA,
        ],
    )(mi_1d, ci_1d, mat_pad, tab_pad)
    return x, length_sorted, idx_unsort


def kernel(matrix, length, core_terms, table):
    return _run(matrix, length, core_terms, table)


# speedup vs baseline: 1.0419x; 1.0419x over previous
import jax
import jax.numpy as jnp
from jax import lax
from jax.experimental import pallas as pl
from jax.experimental.pallas import tpu as pltpu
from jax.experimental.pallas import tpu_sc as plsc

B = 1024
S = 200
M = 64
D = 64
NC = 2
NS = 16
NW = NC * NS
R = B // NW

S0, S1 = 104, 96


def _sort_body(lr_ref, lc_ref, ctf_ref, rank_ref, lsort_ref, isort_ref,
               cts_ref):
    lr = lr_ref[...]
    lc = lc_ref[...]
    ii = lax.broadcasted_iota(jnp.int32, (B, B), 0)
    jj = lax.broadcasted_iota(jnp.int32, (B, B), 1)
    before = (lr > lc) | ((lr == lc) & (jj < ii))
    rank_ref[...] = jnp.sum(before.astype(jnp.int32), axis=1, keepdims=True)
    before2 = (lc > lr) | ((lc == lr) & (ii < jj))
    rank_row = jnp.sum(before2.astype(jnp.int32), axis=0, keepdims=True)
    sel = ii == rank_row
    isort_ref[...] = jnp.sum(jnp.where(sel, jj, 0), axis=1, keepdims=True)
    lsort_ref[...] = jnp.sum(jnp.where(sel, lr, 0), axis=1, keepdims=True)
    p = sel.astype(jnp.float32)
    cts = jax.lax.dot(p, ctf_ref[...], precision=jax.lax.Precision.HIGHEST)
    cts_ref[...] = cts.astype(jnp.int32)


def _sort_tc(length_flat, ct):
    lr = length_flat.reshape(1, B)
    lc = length_flat.reshape(B, 1)
    ctf = ct.astype(jnp.float32)
    rank, lsorted, isort, cts = pl.pallas_call(
        _sort_body,
        out_shape=[
            jax.ShapeDtypeStruct((B, 1), jnp.int32),
            jax.ShapeDtypeStruct((B, 1), jnp.int32),
            jax.ShapeDtypeStruct((B, 1), jnp.int32),
            jax.ShapeDtypeStruct((B, S), jnp.int32),
        ],
    )(lr, lc, ctf)
    return rank.reshape(B), lsorted.reshape(B), isort, cts


def _sc_body(mi_hbm, ci_hbm, mat_hbm, tab_hbm, out_hbm,
             mi_v, ci_v, m0_v, m1_v, t0_v, t1_v, semm0, semm1, semt0, semt1):
    c = lax.axis_index("c")
    s = lax.axis_index("s")
    wid = s * NC + c
    base = wid * R
    pltpu.sync_copy(mi_hbm.at[pl.ds(base * 8, R * 8)], mi_v)
    pltpu.sync_copy(ci_hbm.at[pl.ds(base * S, R * S)], ci_v)

    mbufs = (m0_v, m1_v)
    tbufs = (t0_v, t1_v)
    msems = (semm0, semm1)
    tsems = (semt0, semt1)

    def issue(r, b):
        o = r * S
        pltpu.async_copy(mat_hbm.at[mi_v.at[pl.ds(r * 8, 1)]], mbufs[b],
                         msems[b])
        pltpu.async_copy(tab_hbm.at[ci_v.at[pl.ds(o, S0)]],
                         tbufs[b].at[pl.ds(0, S0)], tsems[b])
        pltpu.async_copy(tab_hbm.at[ci_v.at[pl.ds(o + S0, S1)]],
                         tbufs[b].at[pl.ds(S0, S1)], tsems[b])

    def drain(r, b):
        o = r * S
        pltpu.make_async_copy(mat_hbm.at[mi_v.at[pl.ds(r * 8, 1)]], mbufs[b],
                              msems[b]).wait()
        pltpu.make_async_copy(tab_hbm.at[ci_v.at[pl.ds(o, S0)]],
                              tbufs[b].at[pl.ds(0, S0)], tsems[b]).wait()
        pltpu.make_async_copy(tab_hbm.at[ci_v.at[pl.ds(o + S0, S1)]],
                              tbufs[b].at[pl.ds(S0, S1)], tsems[b]).wait()

    def write(r, b):
        k = base + r
        pltpu.sync_copy(mbufs[b].at[0], out_hbm.at[k, :, pl.ds(0, M)])
        pltpu.sync_copy(tbufs[b].at[:, pl.ds(0, D)],
                        out_hbm.at[k, :, pl.ds(M, D)])

    issue(0, 0)

    def pair(p, carry):
        r0 = 2 * p
        r1 = r0 + 1
        drain(r0, 0)
        issue(r1, 1)
        write(r0, 0)
        drain(r1, 1)

        @pl.when(p + 1 < R // 2)
        def _():
            issue(r0 + 2, 0)

        write(r1, 1)
        return carry

    lax.fori_loop(0, R // 2, pair, 0)


@jax.jit
def _run(matrix, length, core_terms, table):
    length_flat = length.reshape(B)
    ct = core_terms.reshape(B, S)
    idx_unsort, length_sorted, isort, cts = _sort_tc(length_flat, ct)

    mi_1d = jnp.pad(isort, ((0, 0), (0, 7))).reshape(B * 8)
    ci_1d = cts.reshape(B * S)
    mat3d = matrix.reshape(B, S, M)
    tab_pad = jnp.pad(table, ((0, 0), (0, 128 - D)))

    mesh = plsc.VectorSubcoreMesh(core_axis_name="c", subcore_axis_name="s")
    x = pl.kernel(
        _sc_body,
        mesh=mesh,
        out_type=jax.ShapeDtypeStruct((B, S, M + D), jnp.float32),
        compiler_params=pltpu.CompilerParams(use_tc_tiling_on_sc=False),
        scratch_types=[
            pltpu.VMEM((R * 8,), jnp.int32),
            pltpu.VMEM((R * S,), jnp.int32),
            pltpu.VMEM((1, S, M), jnp.float32),
            pltpu.VMEM((1, S, M), jnp.float32),
            pltpu.VMEM((S, 128), jnp.float32),
            pltpu.VMEM((S, 128), jnp.float32),
            pltpu.SemaphoreType.DMA,
            pltpu.SemaphoreType.DMA,
            pltpu.SemaphoreType.DMA,
            pltpu.SemaphoreType.DMA,
        ],
    )(mi_1d, ci_1d, mat3d, tab_pad)
    return x, length_sorted, idx_unsort


def kernel(matrix, length, core_terms, table):
    return _run(matrix, length, core_terms, table)
